# baseline (device time: 52284 ns/iter reference)
import jax
import jax.numpy as jnp
from jax import lax
from jax.experimental import pallas as pl
from jax.experimental.pallas import tpu as pltpu

N_DEV = 4
SQ = 1024
SKV = 1024
D_MODEL = 1024
H_PER_SHARD = 8
DH = 128
SCALE = 0.08838834764831843
N_GROUPS = 4
GQ = SQ // N_GROUPS
GK = SKV // N_GROUPS
CHUNK = SQ // 2 // N_DEV


def _perm_rows(a):
    n, c = a.shape
    return a.reshape(N_GROUPS, N_GROUPS, n // 16, c).transpose(1, 0, 2, 3).reshape(n, c)


def kernel(x, Wq, K_ext, V_ext, Wo):
    idx = lax.axis_index("i")
    K = lax.dynamic_slice(
        K_ext, (0, 0, idx * H_PER_SHARD, 0), (1, SKV, H_PER_SHARD, DH)
    ).reshape(SKV, H_PER_SHARD * DH)
    V = lax.dynamic_slice(
        V_ext, (0, 0, idx * H_PER_SHARD, 0), (1, SKV, H_PER_SHARD, DH)
    ).reshape(SKV, H_PER_SHARD * DH)
    x2 = _perm_rows(x.reshape(SQ, D_MODEL)).astype(jnp.bfloat16)
    Kp = _perm_rows(K).astype(jnp.bfloat16)
    Vp = _perm_rows(V).astype(jnp.bfloat16)
    Wq16 = Wq.astype(jnp.bfloat16)
    Wo16 = Wo.astype(jnp.bfloat16)

    def body(x_ref, wq_ref, k_ref, v_ref, wo_ref, out_ref,
             ctx_ref, part_ref, stage, rs16, send_sems, recv_sems):
        my = lax.axis_index("i")
        left = lax.rem(my + N_DEV - 1, N_DEV)
        right = lax.rem(my + 1, N_DEV)

        barrier_sem = pltpu.get_barrier_semaphore()
        for nbr in (left, right):
            pl.semaphore_signal(
                barrier_sem, inc=1,
                device_id=(nbr,), device_id_type=pl.DeviceIdType.MESH,
            )
        pl.semaphore_wait(barrier_sem, 2)

        def compute_chunk(row0):
            qc = jnp.dot(
                x_ref[pl.ds(row0, CHUNK), :], wq_ref[:],
                preferred_element_type=jnp.float32,
            ).astype(jnp.bfloat16)
            g0 = (row0 // GQ) * GQ
            for h in range(H_PER_SHARD):
                kh = k_ref[pl.ds(g0, GK), h * DH:(h + 1) * DH]
                vh = v_ref[pl.ds(g0, GK), h * DH:(h + 1) * DH]
                s = lax.dot_general(
                    qc[:, h * DH:(h + 1) * DH], kh,
                    (((1,), (1,)), ((), ())),
                    preferred_element_type=jnp.float32,
                ) * SCALE
                m = jnp.max(s, axis=1, keepdims=True)
                w = jnp.exp(s - m)
                p = w / jnp.sum(w, axis=1, keepdims=True)
                ctx_ref[:, h * DH:(h + 1) * DH] = jnp.dot(
                    p.astype(jnp.bfloat16), vh,
                    preferred_element_type=jnp.float32,
                ).astype(jnp.bfloat16)
            part_ref[pl.ds(row0, CHUNK), :] = jnp.dot(
                ctx_ref[:], wo_ref[:], preferred_element_type=jnp.float32
            )

        DIRS = ((0, 1, 0), (1, -1, SQ // 2))
        dests = (right, left)

        def rows(base, c):
            return pl.ds(base + c * CHUNK, CHUNK)

        def start_rs(dirn, s):
            rdma = pltpu.make_async_remote_copy(
                src_ref=stage.at[dirn],
                dst_ref=rs16.at[dirn, s],
                send_sem=send_sems.at[dirn, s],
                recv_sem=recv_sems.at[dirn, s],
                device_id=(dests[dirn],),
                device_id_type=pl.DeviceIdType.MESH,
            )
            rdma.start()
            return rdma

        def start_ag(dirn, base, c, t):
            sl = out_ref.at[0, rows(base, c), :]
            rdma = pltpu.make_async_remote_copy(
                src_ref=sl,
                dst_ref=sl,
                send_sem=send_sems.at[dirn, N_DEV - 1 + t],
                recv_sem=recv_sems.at[dirn, N_DEV - 1 + t],
                device_id=(dests[dirn],),
                device_id_type=pl.DeviceIdType.MESH,
            )
            rdma.start()
            return rdma

        rdmas = [None, None]
        for dirn, sigma, base in DIRS:
            compute_chunk(base + my * CHUNK)
            stage[dirn] = part_ref[rows(base, my), :].astype(jnp.bfloat16)
            rdmas[dirn] = start_rs(dirn, 0)
        owned = {}
        for s in range(N_DEV - 1):
            for dirn, sigma, base in DIRS:
                recv_c = lax.rem(my - sigma * (s + 1) + 8, N_DEV)
                compute_chunk(base + recv_c * CHUNK)
            for dirn, sigma, base in DIRS:
                rdmas[dirn].wait()
                recv_c = lax.rem(my - sigma * (s + 1) + 8, N_DEV)
                acc = (rs16[dirn, s].astype(jnp.float32)
                       + part_ref[rows(base, recv_c), :])
                if s < N_DEV - 2:
                    stage[dirn] = acc.astype(jnp.bfloat16)
                else:
                    owned[dirn] = lax.rem(my + sigma + N_DEV, N_DEV)
                    out_ref[0, rows(base, owned[dirn]), :] = (
                        acc.astype(jnp.bfloat16)
                    )
            if s < N_DEV - 2:
                for dirn, _, _ in DIRS:
                    rdmas[dirn] = start_rs(dirn, s + 1)

        for t in range(N_DEV - 1):
            for dirn, sigma, base in DIRS:
                send_c = lax.rem(owned[dirn] - sigma * t + 8, N_DEV)
                rdmas[dirn] = start_ag(dirn, base, send_c, t)
            for dirn, _, _ in DIRS:
                rdmas[dirn].wait()

    out_perm = pl.pallas_call(
        body,
        out_shape=jax.ShapeDtypeStruct((1, SQ, D_MODEL), jnp.bfloat16),
        in_specs=[pl.BlockSpec(memory_space=pltpu.VMEM)] * 5,
        out_specs=pl.BlockSpec(memory_space=pltpu.VMEM),
        scratch_shapes=[
            pltpu.VMEM((CHUNK, H_PER_SHARD * DH), jnp.bfloat16),
            pltpu.VMEM((SQ, D_MODEL), jnp.float32),
            pltpu.VMEM((2, CHUNK, D_MODEL), jnp.bfloat16),
            pltpu.VMEM((2, N_DEV - 1, CHUNK, D_MODEL), jnp.bfloat16),
            pltpu.SemaphoreType.DMA((2, 2 * (N_DEV - 1))),
            pltpu.SemaphoreType.DMA((2, 2 * (N_DEV - 1))),
        ],
        compiler_params=pltpu.CompilerParams(collective_id=0),
    )(x2, Wq16, Kp, Vp, Wo16)

    out = _perm_rows(out_perm.reshape(SQ, D_MODEL)).astype(jnp.float32)
    return out.reshape(1, SQ, D_MODEL)


# device time: 46833 ns/iter; 1.1164x vs baseline; 1.1164x over previous
import jax
import jax.numpy as jnp
from jax import lax
from jax.experimental import pallas as pl
from jax.experimental.pallas import tpu as pltpu

N_DEV = 4
SQ = 1024
SKV = 1024
D_MODEL = 1024
H_PER_SHARD = 8
DH = 128
SCALE = 0.08838834764831843
N_GROUPS = 4
GQ = SQ // N_GROUPS
GK = SKV // N_GROUPS
BLK = 64
CHUNK = SQ // 2 // N_DEV


def _perm_rows(a):
    n, c = a.shape
    return a.reshape(N_GROUPS, N_GROUPS, n // 16, c).transpose(1, 0, 2, 3).reshape(n, c)


def kernel(x, Wq, K_ext, V_ext, Wo):
    x2 = x.reshape(SQ, D_MODEL)

    def body(x_ref, wq_ref, kext_ref, vext_ref, wo_ref, out_ref,
             wq16, wo16, kscr, vscr, ctx_ref, part_ref, stage, rs16,
             kv_sems, send_sems, recv_sems):
        my = lax.axis_index("i")
        left = lax.rem(my + N_DEV - 1, N_DEV)
        right = lax.rem(my + 1, N_DEV)

        barrier_sem = pltpu.get_barrier_semaphore()
        for nbr in (left, right):
            pl.semaphore_signal(
                barrier_sem, inc=1,
                device_id=(nbr,), device_id_type=pl.DeviceIdType.MESH,
            )
        pl.semaphore_wait(barrier_sem, 2)

        h0 = my * H_PER_SHARD
        kcopy = pltpu.make_async_copy(
            kext_ref.at[0, :, pl.ds(h0, H_PER_SHARD), :], kscr, kv_sems.at[0]
        )
        vcopy = pltpu.make_async_copy(
            vext_ref.at[0, :, pl.ds(h0, H_PER_SHARD), :], vscr, kv_sems.at[1]
        )
        kcopy.start()
        vcopy.start()
        wq16[:] = wq_ref[:].astype(jnp.bfloat16)
        wo16[:] = wo_ref[:].astype(jnp.bfloat16)
        kcopy.wait()
        vcopy.wait()

        def compute_chunk(row0):
            g = row0 // GQ
            sub = lax.rem(row0 // CHUNK, 2)
            o0 = 2 * sub
            xq = jnp.concatenate(
                [x_ref[pl.ds((o0 + j) * GQ + g * BLK, BLK), :]
                 for j in range(2)], axis=0,
            ).astype(jnp.bfloat16)
            qc = jnp.dot(
                xq, wq16[:], preferred_element_type=jnp.float32
            ).astype(jnp.bfloat16)
            kq = jnp.concatenate(
                [kscr[pl.ds(o * GK + g * BLK, BLK), :, :]
                 for o in range(N_GROUPS)], axis=0,
            ).astype(jnp.bfloat16).reshape(GK, H_PER_SHARD * DH)
            vq = jnp.concatenate(
                [vscr[pl.ds(o * GK + g * BLK, BLK), :, :]
                 for o in range(N_GROUPS)], axis=0,
            ).astype(jnp.bfloat16).reshape(GK, H_PER_SHARD * DH)
            for h in range(H_PER_SHARD):
                kh = kq[:, h * DH:(h + 1) * DH]
                vh = vq[:, h * DH:(h + 1) * DH]
                s = lax.dot_general(
                    qc[:, h * DH:(h + 1) * DH], kh,
                    (((1,), (1,)), ((), ())),
                    preferred_element_type=jnp.float32,
                ) * SCALE
                m = jnp.max(s, axis=1, keepdims=True)
                w = jnp.exp(s - m)
                p = w / jnp.sum(w, axis=1, keepdims=True)
                ctx_ref[:, h * DH:(h + 1) * DH] = jnp.dot(
                    p.astype(jnp.bfloat16), vh,
                    preferred_element_type=jnp.float32,
                ).astype(jnp.bfloat16)
            part_ref[pl.ds(row0, CHUNK), :] = jnp.dot(
                ctx_ref[:], wo16[:], preferred_element_type=jnp.float32
            )

        DIRS = ((0, 1, 0), (1, -1, SQ // 2))
        dests = (right, left)

        def rows(base, c):
            return pl.ds(base + c * CHUNK, CHUNK)

        def start_rs(dirn, s):
            rdma = pltpu.make_async_remote_copy(
                src_ref=stage.at[dirn],
                dst_ref=rs16.at[dirn, s],
                send_sem=send_sems.at[dirn, s],
                recv_sem=recv_sems.at[dirn, s],
                device_id=(dests[dirn],),
                device_id_type=pl.DeviceIdType.MESH,
            )
            rdma.start()
            return rdma

        def start_ag(dirn, base, c, t):
            sl = out_ref.at[0, rows(base, c), :]
            rdma = pltpu.make_async_remote_copy(
                src_ref=sl,
                dst_ref=sl,
                send_sem=send_sems.at[dirn, N_DEV - 1 + t],
                recv_sem=recv_sems.at[dirn, N_DEV - 1 + t],
                device_id=(dests[dirn],),
                device_id_type=pl.DeviceIdType.MESH,
            )
            rdma.start()
            return rdma

        rdmas = [None, None]
        for dirn, sigma, base in DIRS:
            compute_chunk(base + my * CHUNK)
            stage[dirn] = part_ref[rows(base, my), :].astype(jnp.bfloat16)
            rdmas[dirn] = start_rs(dirn, 0)
        owned = {}
        for s in range(N_DEV - 1):
            for dirn, sigma, base in DIRS:
                recv_c = lax.rem(my - sigma * (s + 1) + 8, N_DEV)
                compute_chunk(base + recv_c * CHUNK)
            for dirn, sigma, base in DIRS:
                rdmas[dirn].wait()
                recv_c = lax.rem(my - sigma * (s + 1) + 8, N_DEV)
                acc = (rs16[dirn, s].astype(jnp.float32)
                       + part_ref[rows(base, recv_c), :])
                if s < N_DEV - 2:
                    stage[dirn] = acc.astype(jnp.bfloat16)
                else:
                    owned[dirn] = lax.rem(my + sigma + N_DEV, N_DEV)
                    out_ref[0, rows(base, owned[dirn]), :] = (
                        acc.astype(jnp.bfloat16)
                    )
            if s < N_DEV - 2:
                for dirn, _, _ in DIRS:
                    rdmas[dirn] = start_rs(dirn, s + 1)

        for t in range(N_DEV - 1):
            for dirn, sigma, base in DIRS:
                send_c = lax.rem(owned[dirn] - sigma * t + 8, N_DEV)
                rdmas[dirn] = start_ag(dirn, base, send_c, t)
            for dirn, _, _ in DIRS:
                rdmas[dirn].wait()

    out_perm = pl.pallas_call(
        body,
        out_shape=jax.ShapeDtypeStruct((1, SQ, D_MODEL), jnp.bfloat16),
        in_specs=[
            pl.BlockSpec(memory_space=pltpu.VMEM),
            pl.BlockSpec(memory_space=pltpu.VMEM),
            pl.BlockSpec(memory_space=pltpu.MemorySpace.HBM),
            pl.BlockSpec(memory_space=pltpu.MemorySpace.HBM),
            pl.BlockSpec(memory_space=pltpu.VMEM),
        ],
        out_specs=pl.BlockSpec(memory_space=pltpu.VMEM),
        scratch_shapes=[
            pltpu.VMEM((D_MODEL, D_MODEL), jnp.bfloat16),
            pltpu.VMEM((D_MODEL, D_MODEL), jnp.bfloat16),
            pltpu.VMEM((SKV, H_PER_SHARD, DH), jnp.float32),
            pltpu.VMEM((SKV, H_PER_SHARD, DH), jnp.float32),
            pltpu.VMEM((CHUNK, H_PER_SHARD * DH), jnp.bfloat16),
            pltpu.VMEM((SQ, D_MODEL), jnp.float32),
            pltpu.VMEM((2, CHUNK, D_MODEL), jnp.bfloat16),
            pltpu.VMEM((2, N_DEV - 1, CHUNK, D_MODEL), jnp.bfloat16),
            pltpu.SemaphoreType.DMA((2,)),
            pltpu.SemaphoreType.DMA((2, 2 * (N_DEV - 1))),
            pltpu.SemaphoreType.DMA((2, 2 * (N_DEV - 1))),
        ],
        compiler_params=pltpu.CompilerParams(collective_id=0),
    )(x2, Wq, K_ext, V_ext, Wo)

    out = _perm_rows(out_perm.reshape(SQ, D_MODEL)).astype(jnp.float32)
    return out.reshape(1, SQ, D_MODEL)
